# baseline (speedup 1.0000x reference)
import jax
import jax.numpy as jnp
from jax.experimental import pallas as pl
from jax.experimental.pallas import tpu as pltpu

_TOKENS = 32768
_HIDDEN = 768
_EXPERTS = 64
_BT = 4096


def _gate_kernel(x_ref, w_ref, scores_ref, idx_ref):
    x = x_ref[...]
    w = w_ref[...]
    logits = jax.lax.dot_general(
        x, w, (((1,), (1,)), ((), ())), preferred_element_type=jnp.float32
    )
    lane = jax.lax.broadcasted_iota(jnp.int32, logits.shape, 1).astype(
        jnp.float32
    )
    big = jnp.float32(_EXPERTS)
    m1 = jnp.max(logits, axis=-1, keepdims=True)
    i1 = jnp.min(jnp.where(logits == m1, lane, big), axis=-1, keepdims=True)
    masked = jnp.where(lane == i1, -jnp.inf, logits)
    m2 = jnp.max(masked, axis=-1, keepdims=True)
    i2 = jnp.min(jnp.where(masked == m2, lane, big), axis=-1, keepdims=True)
    e = jnp.exp(m2 - m1)
    s1 = 1.0 / (1.0 + e)
    col = jax.lax.broadcasted_iota(jnp.int32, (logits.shape[0], 2), 1)
    scores_ref[...] = jnp.where(col == 0, s1, 1.0 - s1)
    idx_ref[...] = jnp.where(col == 0, i1, i2).astype(jnp.int32)


def kernel(x, W):
    grid = (_TOKENS // _BT,)
    scores, idx = pl.pallas_call(
        _gate_kernel,
        grid=grid,
        in_specs=[
            pl.BlockSpec((_BT, _HIDDEN), lambda i: (i, 0)),
            pl.BlockSpec((_EXPERTS, _HIDDEN), lambda i: (0, 0)),
        ],
        out_specs=[
            pl.BlockSpec((_BT, 2), lambda i: (i, 0)),
            pl.BlockSpec((_BT, 2), lambda i: (i, 0)),
        ],
        out_shape=[
            jax.ShapeDtypeStruct((_TOKENS, 2), jnp.float32),
            jax.ShapeDtypeStruct((_TOKENS, 2), jnp.int32),
        ],
        compiler_params=pltpu.CompilerParams(
            dimension_semantics=("arbitrary",),
        ),
    )(x, W)
    return scores, idx


# speedup vs baseline: 1.0016x; 1.0016x over previous
import jax
import jax.numpy as jnp
from jax.experimental import pallas as pl
from jax.experimental.pallas import tpu as pltpu

_BT = 4096


def _gate_kernel(x_ref, w_ref, scores_ref, idx_ref):
    x = x_ref[...]
    w = w_ref[...]
    logits = jax.lax.dot_general(
        x, w, (((1,), (1,)), ((), ())), preferred_element_type=jnp.float32
    )
    lane = jax.lax.broadcasted_iota(jnp.int32, logits.shape, 1).astype(
        jnp.float32
    )
    big = jnp.float32(logits.shape[-1])
    m1 = jnp.max(logits, axis=-1, keepdims=True)
    i1 = jnp.min(jnp.where(logits == m1, lane, big), axis=-1, keepdims=True)
    masked = jnp.where(lane == i1, -jnp.inf, logits)
    m2 = jnp.max(masked, axis=-1, keepdims=True)
    i2 = jnp.min(jnp.where(masked == m2, lane, big), axis=-1, keepdims=True)
    e = jnp.exp(m2 - m1)
    s1 = 1.0 / (1.0 + e)
    col = jax.lax.broadcasted_iota(jnp.int32, (logits.shape[0], 2), 1)
    scores_ref[...] = jnp.where(col == 0, s1, 1.0 - s1)
    idx_ref[...] = jnp.where(col == 0, i1, i2).astype(jnp.int32)


def kernel(x, W):
    tokens, hidden = x.shape
    experts = W.shape[0]
    bt = _BT if tokens % _BT == 0 else tokens
    scores, idx = pl.pallas_call(
        _gate_kernel,
        grid=(tokens // bt,),
        in_specs=[
            pl.BlockSpec((bt, hidden), lambda i: (i, 0)),
            pl.BlockSpec((experts, hidden), lambda i: (0, 0)),
        ],
        out_specs=[
            pl.BlockSpec((bt, 2), lambda i: (i, 0)),
            pl.BlockSpec((bt, 2), lambda i: (i, 0)),
        ],
        out_shape=[
            jax.ShapeDtypeStruct((tokens, 2), jnp.float32),
            jax.ShapeDtypeStruct((tokens, 2), jnp.int32),
        ],
        compiler_params=pltpu.CompilerParams(
            dimension_semantics=("arbitrary",),
        ),
    )(x, W)
    return scores, idx
